# Initial kernel scaffold; baseline (speedup 1.0000x reference)
import jax
import jax.numpy as jnp
from jax.experimental import pallas as pl
from jax.experimental.pallas import tpu as pltpu

B = 128
D = 2048
H = D // 2
CACHE = 16384


def _lf_kernel(x_ref, pre_ref, out_idx_ref, idx_row_ref, idx_col_ref,
               w1_ref, b1_ref, w2_ref, b2_ref, nw_ref,
               lf1_ref, lf2_ref,
               out_ref, new1_ref, new2_ref,
               g1_ref, g2_ref, v1_ref, v2_ref, gsem, ssem):
    def gather_start(i, _):
        j = pre_ref[0, i]
        pltpu.make_async_copy(lf1_ref.at[j], g1_ref.at[i], gsem).start()
        pltpu.make_async_copy(lf2_ref.at[j], g2_ref.at[i], gsem).start()
        return 0

    jax.lax.fori_loop(0, B, gather_start, 0)

    def gather_wait(i, _):
        j = pre_ref[0, i]
        pltpu.make_async_copy(lf1_ref.at[j], g1_ref.at[i], gsem).wait()
        pltpu.make_async_copy(lf2_ref.at[j], g2_ref.at[i], gsem).wait()
        return 0

    jax.lax.fori_loop(0, B, gather_wait, 0)

    x = x_ref[...]

    col = idx_col_ref[...]
    row = idx_row_ref[...]
    eq = col == row
    jj = jax.lax.broadcasted_iota(jnp.int32, (B, B), 1)
    last = jnp.max(jnp.where(eq, jj, -1), axis=1, keepdims=True)
    p = (jj == last).astype(jnp.float32)

    v1_ref[...] = jnp.dot(p, x, preferred_element_type=jnp.float32)

    def scat1_start(i, _):
        pltpu.make_async_copy(v1_ref.at[i], new1_ref.at[out_idx_ref[0, i]],
                              ssem).start()
        return 0

    jax.lax.fori_loop(0, B, scat1_start, 0)

    g1 = g1_ref[...]
    out1 = (jnp.dot(g1, w1_ref[:, :H], preferred_element_type=jnp.float32)
            + jnp.dot(x, w1_ref[:, H:], preferred_element_type=jnp.float32)
            + b1_ref[...])

    v2_ref[...] = jnp.dot(p, out1, preferred_element_type=jnp.float32)

    def scat2_start(i, _):
        pltpu.make_async_copy(v2_ref.at[i], new2_ref.at[out_idx_ref[0, i]],
                              ssem).start()
        return 0

    jax.lax.fori_loop(0, B, scat2_start, 0)

    g2 = g2_ref[...]
    out2 = (jnp.dot(g2, w2_ref[:, :D], preferred_element_type=jnp.float32)
            + jnp.dot(out1, w2_ref[:, D:], preferred_element_type=jnp.float32)
            + b2_ref[...])
    out3 = out2 + x
    var = jnp.mean(out3 * out3, axis=-1, keepdims=True)
    out_ref[...] = out3 * jax.lax.rsqrt(var + 1e-6) * nw_ref[...]

    def scat_wait(i, _):
        pltpu.make_async_copy(v1_ref.at[i], new1_ref.at[out_idx_ref[0, i]],
                              ssem).wait()
        pltpu.make_async_copy(v2_ref.at[i], new2_ref.at[out_idx_ref[0, i]],
                              ssem).wait()
        return 0

    jax.lax.fori_loop(0, B, scat_wait, 0)


def kernel(inputs, pre_lf_indexs, out_lf_indexs, input_lf_loc, out_lf_loc,
           inputs_loc, outputs_loc, kv_cache, conv1_weight, conv1_bias,
           conv2_weight, conv2_bias, lf1_caches, lf2_caches, norm_weight):
    pre_i32 = pre_lf_indexs.astype(jnp.int32)
    out_i32 = out_lf_indexs.astype(jnp.int32)
    pre_sm = pre_i32.reshape(1, B)
    out_sm = out_i32.reshape(1, B)
    idx_row = out_i32.reshape(1, B)
    idx_col = out_i32.reshape(B, 1)

    vmem = pl.BlockSpec(memory_space=pltpu.MemorySpace.VMEM)
    smem = pl.BlockSpec(memory_space=pltpu.MemorySpace.SMEM)
    anym = pl.BlockSpec(memory_space=pltpu.MemorySpace.ANY)

    out, new1, new2 = pl.pallas_call(
        _lf_kernel,
        out_shape=[
            jax.ShapeDtypeStruct((B, D), jnp.float32),
            jax.ShapeDtypeStruct((CACHE, D), jnp.float32),
            jax.ShapeDtypeStruct((CACHE, H), jnp.float32),
        ],
        in_specs=[vmem, smem, smem, vmem, vmem,
                  vmem, vmem, vmem, vmem, vmem,
                  anym, anym],
        out_specs=[vmem, anym, anym],
        scratch_shapes=[
            pltpu.VMEM((B, D), jnp.float32),
            pltpu.VMEM((B, H), jnp.float32),
            pltpu.VMEM((B, D), jnp.float32),
            pltpu.VMEM((B, H), jnp.float32),
            pltpu.SemaphoreType.DMA,
            pltpu.SemaphoreType.DMA,
        ],
        input_output_aliases={10: 1, 11: 2},
        compiler_params=pltpu.CompilerParams(
            vmem_limit_bytes=100 * 1024 * 1024,
        ),
    )(inputs, pre_sm, out_sm, idx_row, idx_col,
      conv1_weight, conv1_bias.reshape(1, H),
      conv2_weight, conv2_bias.reshape(1, D),
      norm_weight.reshape(1, D),
      lf1_caches, lf2_caches)

    return out, new1, new2


# speedup vs baseline: 1.3816x; 1.3816x over previous
import jax
import jax.numpy as jnp
from jax.experimental import pallas as pl
from jax.experimental.pallas import tpu as pltpu

B = 128
D = 2048
H = D // 2
CACHE = 16384


def _lf_kernel(x_ref, pre_ref, out_idx_ref, idx_row_ref, idx_col_ref,
               w1_ref, b1_ref, w2_ref, b2_ref, nw_ref,
               lf1_ref, lf2_ref,
               out_ref, new1_ref, new2_ref,
               g1_ref, g2_ref, v1_ref, v2_ref, gsem, ssem):
    def gather_start(i, _):
        j = pre_ref[0, i]
        pltpu.make_async_copy(lf1_ref.at[j], g1_ref.at[i], gsem).start()
        pltpu.make_async_copy(lf2_ref.at[j], g2_ref.at[i], gsem).start()
        return 0

    jax.lax.fori_loop(0, B, gather_start, 0)

    def gather_wait(i, _):
        j = pre_ref[0, i]
        pltpu.make_async_copy(lf1_ref.at[j], g1_ref.at[i], gsem).wait()
        pltpu.make_async_copy(lf2_ref.at[j], g2_ref.at[i], gsem).wait()
        return 0

    jax.lax.fori_loop(0, B, gather_wait, 0)

    x = x_ref[...]

    col = idx_col_ref[...]
    row = idx_row_ref[...]
    eq = col == row
    jj = jax.lax.broadcasted_iota(jnp.int32, (B, B), 1)
    last = jnp.max(jnp.where(eq, jj, -1), axis=1, keepdims=True)
    p = (jj == last).astype(jnp.float32)

    v1_ref[...] = jnp.dot(p, x, preferred_element_type=jnp.float32)

    def scat1_start(i, _):
        pltpu.make_async_copy(v1_ref.at[i], new1_ref.at[out_idx_ref[0, i]],
                              ssem).start()
        return 0

    jax.lax.fori_loop(0, B, scat1_start, 0)

    g1 = g1_ref[...]
    out1 = (jnp.dot(g1, w1_ref[:, :H], preferred_element_type=jnp.float32)
            + jnp.dot(x, w1_ref[:, H:], preferred_element_type=jnp.float32)
            + b1_ref[...])

    v2_ref[...] = jnp.dot(p, out1, preferred_element_type=jnp.float32)

    def scat2_start(i, _):
        pltpu.make_async_copy(v2_ref.at[i], new2_ref.at[out_idx_ref[0, i]],
                              ssem).start()
        return 0

    jax.lax.fori_loop(0, B, scat2_start, 0)

    g2 = g2_ref[...]
    out2 = (jnp.dot(g2, w2_ref[:, :D], preferred_element_type=jnp.float32)
            + jnp.dot(out1, w2_ref[:, D:], preferred_element_type=jnp.float32)
            + b2_ref[...])
    out3 = out2 + x
    var = jnp.mean(out3 * out3, axis=-1, keepdims=True)
    out_ref[...] = out3 * jax.lax.rsqrt(var + 1e-6) * nw_ref[...]

    def scat_wait(i, _):
        pltpu.make_async_copy(v1_ref.at[i], new1_ref.at[out_idx_ref[0, i]],
                              ssem).wait()
        pltpu.make_async_copy(v2_ref.at[i], new2_ref.at[out_idx_ref[0, i]],
                              ssem).wait()
        return 0

    jax.lax.fori_loop(0, B, scat_wait, 0)


def kernel(inputs, pre_lf_indexs, out_lf_indexs, input_lf_loc, out_lf_loc,
           inputs_loc, outputs_loc, kv_cache, conv1_weight, conv1_bias,
           conv2_weight, conv2_bias, lf1_caches, lf2_caches, norm_weight):
    pre_i32 = pre_lf_indexs.astype(jnp.int32)
    out_i32 = out_lf_indexs.astype(jnp.int32)
    pre_sm = pre_i32.reshape(1, B)
    out_sm = out_i32.reshape(1, B)
    idx_row = out_i32.reshape(1, B)
    idx_col = out_i32.reshape(B, 1)

    vmem = pl.BlockSpec(memory_space=pltpu.MemorySpace.VMEM)
    smem = pl.BlockSpec(memory_space=pltpu.MemorySpace.SMEM)
    anym = pl.BlockSpec(memory_space=pl.ANY)

    out, new1, new2 = pl.pallas_call(
        _lf_kernel,
        out_shape=[
            jax.ShapeDtypeStruct((B, D), jnp.float32),
            jax.ShapeDtypeStruct((CACHE, D), jnp.float32),
            jax.ShapeDtypeStruct((CACHE, H), jnp.float32),
        ],
        in_specs=[vmem, smem, smem, vmem, vmem,
                  vmem, vmem, vmem, vmem, vmem,
                  anym, anym],
        out_specs=[vmem, anym, anym],
        scratch_shapes=[
            pltpu.VMEM((B, D), jnp.float32),
            pltpu.VMEM((B, H), jnp.float32),
            pltpu.VMEM((B, D), jnp.float32),
            pltpu.VMEM((B, H), jnp.float32),
            pltpu.SemaphoreType.DMA,
            pltpu.SemaphoreType.DMA,
        ],
        input_output_aliases={10: 1, 11: 2},
        compiler_params=pltpu.CompilerParams(
            vmem_limit_bytes=100 * 1024 * 1024,
        ),
    )(inputs, pre_sm, out_sm, idx_row, idx_col,
      conv1_weight, conv1_bias.reshape(1, H),
      conv2_weight, conv2_bias.reshape(1, D),
      norm_weight.reshape(1, D),
      lf1_caches, lf2_caches)

    return out, new1, new2
